# Initial kernel scaffold; baseline (speedup 1.0000x reference)
import jax
import jax.numpy as jnp
from jax.experimental import pallas as pl


def kernel(x):
    raise NotImplementedError("write your pallas kernel here")


# speedup vs baseline: 1.8770x; 1.8770x over previous
import functools

import jax
import jax.numpy as jnp
from jax import lax
from jax.experimental import pallas as pl
from jax.experimental.pallas import tpu as pltpu
from jax.experimental.pallas import tpu_sc as plsc

NUM_CLASSES = 1000
ROWS = 4096
COLS = 20
N = ROWS * COLS
L = 16
NW = 32
PER_W = N // NW
B = 32
NCHUNK = PER_W // B
CHUNK_W = B * NUM_CLASSES


def _scatter_chunk(buf, idx_v, c, value):
    ones = jnp.full((L,), value, jnp.float32)
    for j in range(B // L):
        pos = c * B + j * L
        ids = idx_v[pl.ds(pos, L)]
        valid = (ids >= 0) & (ids < NUM_CLASSES)
        rows = lax.iota(jnp.int32, L) + j * L
        flat = rows * NUM_CLASSES + jnp.where(valid, ids, 0)
        plsc.store_scatter(buf, [flat], ones, mask=valid)


def _make_sc_one_hot():
    mesh = plsc.VectorSubcoreMesh(core_axis_name="c", subcore_axis_name="s")

    @functools.partial(
        pl.kernel,
        mesh=mesh,
        compiler_params=pltpu.CompilerParams(needs_layout_passes=False),
        out_type=jax.ShapeDtypeStruct((N * NUM_CLASSES,), jnp.float32),
        scratch_types=[
            pltpu.VMEM((PER_W,), jnp.int32),
            pltpu.VMEM((CHUNK_W,), jnp.float32),
            pltpu.VMEM((CHUNK_W,), jnp.float32),
            pltpu.SemaphoreType.DMA,
            pltpu.SemaphoreType.DMA,
        ],
    )
    def k(x_hbm, out_hbm, idx_v, buf0, buf1, sem0, sem1):
        wid = lax.axis_index("s") * 2 + lax.axis_index("c")
        row_base = wid * PER_W

        pltpu.sync_copy(x_hbm.at[pl.ds(row_base, PER_W)], idx_v)

        z = jnp.zeros((L,), jnp.float32)

        def zbody(i, _):
            buf0[pl.ds(i * L, L)] = z
            buf1[pl.ds(i * L, L)] = z
            return 0

        lax.fori_loop(0, CHUNK_W // L, zbody, 0)

        bufs = (buf0, buf1)
        sems = (sem0, sem1)

        def fire(c, buf, sem):
            off = (row_base + c * B) * NUM_CLASSES
            pltpu.async_copy(buf, out_hbm.at[pl.ds(off, CHUNK_W)], sem)

        def drain(c, buf, sem):
            off = (row_base + c * B) * NUM_CLASSES
            pltpu.make_async_copy(buf, out_hbm.at[pl.ds(off, CHUNK_W)], sem).wait()

        for b in range(2):
            _scatter_chunk(bufs[b], idx_v, b, 1.0)
            fire(b, bufs[b], sems[b])

        def body(g, _):
            for b in range(2):
                c = g + b
                drain(c - 2, bufs[b], sems[b])
                _scatter_chunk(bufs[b], idx_v, c - 2, 0.0)
                _scatter_chunk(bufs[b], idx_v, c, 1.0)
                fire(c, bufs[b], sems[b])
            return 0

        lax.fori_loop(1, NCHUNK // 2, lambda g, s: body(g * 2, s), 0)

        for b in range(2):
            drain(NCHUNK - 2 + b, bufs[b], sems[b])

    return k


_sc_one_hot = _make_sc_one_hot()


@jax.jit
def kernel(x):
    xf = x.reshape(-1).astype(jnp.int32)
    out = _sc_one_hot(xf)
    return out.reshape(ROWS, COLS, NUM_CLASSES)
